# Initial kernel scaffold; baseline (speedup 1.0000x reference)
import functools

import jax
import jax.numpy as jnp
from jax import lax
from jax.experimental import pallas as pl
from jax.experimental.pallas import tpu as pltpu
from jax.experimental.pallas import tpu_sc as plsc

N = 50000
E = 800000
IN_DIM = 128
HID = 64
UP_DIM = 768

NPAD = 50176
STRIPE = NPAD // 16
ZCH = 784

EPB = 128
RPT = 392
EPAD = 16 * RPT * EPB
ROWS = EPAD // EPB
K = 8
NBLK = RPT // K

BLK = 512
GRID = NPAD // BLK

_mesh = plsc.VectorSubcoreMesh(core_axis_name="c", subcore_axis_name="s")


def _deg_body(src_rs, dst_rs, ones_hbm, zeros_hbm, out_src, out_dst,
              idx_v, ones_v, hist):
    cid = lax.axis_index("c")
    tid = lax.axis_index("s")

    def run(eidx, out_ref):
        for j in range(STRIPE // ZCH):
            pltpu.sync_copy(zeros_hbm, hist.at[pl.ds(tid * STRIPE + j * ZCH, ZCH)])
        pltpu.sync_copy(ones_hbm, ones_v)
        plsc.subcore_barrier()
        base = tid * RPT

        def body(blk, carry):
            off = base + blk * K
            pltpu.sync_copy(eidx.at[pl.ds(off, K)], idx_v)
            for k in range(K):
                pltpu.sync_copy(ones_v, hist.at[idx_v.at[k]], add=True)
            return carry

        lax.fori_loop(0, NBLK, body, 0)
        plsc.subcore_barrier()
        sl = pl.ds(tid * STRIPE, STRIPE)
        pltpu.sync_copy(hist.at[sl], out_ref.at[sl])

    @pl.when(cid == 0)
    def _():
        run(src_rs, out_src)

    @pl.when(cid == 1)
    def _():
        run(dst_rs, out_dst)


_deg_call = functools.partial(
    pl.kernel,
    out_type=(
        jax.ShapeDtypeStruct((NPAD, 16), jnp.float32),
        jax.ShapeDtypeStruct((NPAD, 16), jnp.float32),
    ),
    mesh=_mesh,
    scratch_types=[
        pltpu.VMEM((K, EPB), jnp.int32),
        pltpu.VMEM((EPB, 16), jnp.float32),
        pltpu.VMEM_SHARED((NPAD, 16), jnp.float32),
    ],
)(_deg_body)


def _agg_body(src_rs, dst_rs, h0, h1, zeros_hbm, out0, out1,
              sidx, didx, rows, sem, acc):
    cid = lax.axis_index("c")
    tid = lax.axis_index("s")

    def run(h_ref, out_ref):
        for j in range(STRIPE // ZCH):
            pltpu.sync_copy(zeros_hbm, acc.at[pl.ds(tid * STRIPE + j * ZCH, ZCH)])
        plsc.subcore_barrier()
        base = tid * RPT

        def body(blk, carry):
            off = base + blk * K
            pltpu.sync_copy(src_rs.at[pl.ds(off, K)], sidx)
            pltpu.sync_copy(dst_rs.at[pl.ds(off, K)], didx)
            descs = [
                pltpu.async_copy(h_ref.at[sidx.at[k]], rows.at[k], sem)
                for k in range(K)
            ]
            for d in descs:
                d.wait()
            for k in range(K):
                pltpu.sync_copy(rows.at[k], acc.at[didx.at[k]], add=True)
            return carry

        lax.fori_loop(0, NBLK, body, 0)
        plsc.subcore_barrier()
        sl = pl.ds(tid * STRIPE, STRIPE)
        pltpu.sync_copy(acc.at[sl], out_ref.at[sl])

    @pl.when(cid == 0)
    def _():
        run(h0, out0)

    @pl.when(cid == 1)
    def _():
        run(h1, out1)


_agg_call = functools.partial(
    pl.kernel,
    out_type=(
        jax.ShapeDtypeStruct((NPAD, 32), jnp.float32),
        jax.ShapeDtypeStruct((NPAD, 32), jnp.float32),
    ),
    mesh=_mesh,
    scratch_types=[
        pltpu.VMEM((K, EPB), jnp.int32),
        pltpu.VMEM((K, EPB), jnp.int32),
        pltpu.VMEM((K, EPB, 32), jnp.float32),
        pltpu.SemaphoreType.DMA,
        pltpu.VMEM_SHARED((NPAD, 32), jnp.float32),
    ],
)(_agg_body)


def _down_body(x_ref, w_ref, b_ref, d_ref, h0_ref, h1_ref):
    h = jnp.dot(x_ref[...], w_ref[...],
                preferred_element_type=jnp.float32,
                precision=lax.Precision.HIGHEST) + b_ref[...]
    norm = lax.rsqrt(jnp.maximum(d_ref[:, :1], 1.0))
    h = h * norm
    h0_ref[...] = h[:, :32]
    h1_ref[...] = h[:, 32:]


def _down_call(x, w, b, deg):
    return pl.pallas_call(
        _down_body,
        grid=(GRID,),
        in_specs=[
            pl.BlockSpec((BLK, IN_DIM), lambda i: (i, 0)),
            pl.BlockSpec((IN_DIM, HID), lambda i: (0, 0)),
            pl.BlockSpec((1, HID), lambda i: (0, 0)),
            pl.BlockSpec((BLK, 16), lambda i: (i, 0)),
        ],
        out_specs=(
            pl.BlockSpec((BLK, 32), lambda i: (i, 0)),
            pl.BlockSpec((BLK, 32), lambda i: (i, 0)),
        ),
        out_shape=(
            jax.ShapeDtypeStruct((NPAD, 32), jnp.float32),
            jax.ShapeDtypeStruct((NPAD, 32), jnp.float32),
        ),
    )(x, w, b, deg)


def _up_body(a0_ref, a1_ref, d_ref, wg_ref, bg_ref, wu_ref, bu_ref, o_ref):
    a = jnp.concatenate([a0_ref[...], a1_ref[...]], axis=1)
    norm = lax.rsqrt(jnp.maximum(d_ref[:, :1], 1.0))
    a = a * norm
    g = jnp.dot(a, wg_ref[...],
                preferred_element_type=jnp.float32,
                precision=lax.Precision.HIGHEST) + bg_ref[...]
    o_ref[...] = jnp.dot(g, wu_ref[...],
                         preferred_element_type=jnp.float32,
                         precision=lax.Precision.HIGHEST) + bu_ref[...]


def _up_call(a0, a1, deg, wg, bg, wu, bu):
    return pl.pallas_call(
        _up_body,
        grid=(GRID,),
        in_specs=[
            pl.BlockSpec((BLK, 32), lambda i: (i, 0)),
            pl.BlockSpec((BLK, 32), lambda i: (i, 0)),
            pl.BlockSpec((BLK, 16), lambda i: (i, 0)),
            pl.BlockSpec((HID, HID), lambda i: (0, 0)),
            pl.BlockSpec((1, HID), lambda i: (0, 0)),
            pl.BlockSpec((HID, UP_DIM), lambda i: (0, 0)),
            pl.BlockSpec((1, UP_DIM), lambda i: (0, 0)),
        ],
        out_specs=pl.BlockSpec((BLK, UP_DIM), lambda i: (i, 0)),
        out_shape=jax.ShapeDtypeStruct((NPAD, UP_DIM), jnp.float32),
    )(a0, a1, deg, wg, bg, wu, bu)


def kernel(features, edge_index, W_down, b_down, W_gnn, b_gnn, W_up, b_up):
    src = edge_index[0]
    dst = edge_index[1]
    npad_extra = NPAD - N
    pad_idx = N + (jnp.arange(EPAD - E, dtype=jnp.int32) % npad_extra)
    src_rs = jnp.concatenate([src, pad_idx]).reshape(ROWS, EPB)
    dst_rs = jnp.concatenate([dst, pad_idx]).reshape(ROWS, EPB)

    ones16 = jnp.ones((EPB, 16), jnp.float32)
    zeros16 = jnp.zeros((ZCH, 16), jnp.float32)
    zeros32 = jnp.zeros((ZCH, 32), jnp.float32)

    deg_src, deg_dst = _deg_call(src_rs, dst_rs, ones16, zeros16)

    feats_pad = jnp.pad(features, ((0, npad_extra), (0, 0)))
    h0, h1 = _down_call(feats_pad, W_down, b_down.reshape(1, HID), deg_src)

    agg0, agg1 = _agg_call(src_rs, dst_rs, h0, h1, zeros32)

    out = _up_call(agg0, agg1, deg_dst, W_gnn, b_gnn.reshape(1, HID),
                   W_up, b_up.reshape(1, UP_DIM))
    return out[:N]


# speedup vs baseline: 5.6215x; 5.6215x over previous
import functools

import jax
import jax.numpy as jnp
from jax import lax
from jax.experimental import pallas as pl
from jax.experimental.pallas import tpu as pltpu
from jax.experimental.pallas import tpu_sc as plsc

N = 50000
E = 800000
IN_DIM = 128
HID = 64
UP_DIM = 768

NPAD = 50176
STRIPE = NPAD // 16
ZCH = 784

EPB = 128
RPT = 392
EPAD = 16 * RPT * EPB
ROWS = EPAD // EPB
K = 4
NBLK = RPT // K

BLK = 512
GRID = NPAD // BLK

_mesh = plsc.VectorSubcoreMesh(core_axis_name="c", subcore_axis_name="s")
_sc_params = pltpu.CompilerParams(use_tc_tiling_on_sc=False)


def _deg_body(src_rs, dst_rs, ones_hbm, zeros_hbm, out_src, out_dst,
              idx_v, ones_v, hist):
    cid = lax.axis_index("c")
    tid = lax.axis_index("s")

    def run(eidx, out_ref):
        for j in range(STRIPE // ZCH):
            pltpu.sync_copy(zeros_hbm, hist.at[pl.ds(tid * STRIPE + j * ZCH, ZCH)])
        pltpu.sync_copy(ones_hbm, ones_v)
        plsc.subcore_barrier()
        base = tid * RPT

        def body(blk, carry):
            off = base + blk * K
            pltpu.sync_copy(eidx.at[pl.ds(off, K)], idx_v)
            for k in range(K):
                pltpu.sync_copy(ones_v, hist.at[idx_v.at[k]], add=True)
            return carry

        lax.fori_loop(0, NBLK, body, 0)
        plsc.subcore_barrier()
        sl = pl.ds(tid * STRIPE, STRIPE)
        pltpu.sync_copy(hist.at[sl], out_ref.at[sl])

    @pl.when(cid == 0)
    def _():
        run(src_rs, out_src)

    @pl.when(cid == 1)
    def _():
        run(dst_rs, out_dst)


_deg_call = functools.partial(
    pl.kernel,
    out_type=(
        jax.ShapeDtypeStruct((NPAD, 16), jnp.float32),
        jax.ShapeDtypeStruct((NPAD, 16), jnp.float32),
    ),
    mesh=_mesh,
    scratch_types=[
        pltpu.VMEM((K, EPB), jnp.int32),
        pltpu.VMEM((EPB, 16), jnp.float32),
        pltpu.VMEM_SHARED((NPAD, 16), jnp.float32),
    ],
    compiler_params=_sc_params,
)(_deg_body)


def _agg_body(src_rs, dst_rs, h0, h1, zeros_hbm, out0, out1,
              sidx, didx, rows, sem, acc):
    cid = lax.axis_index("c")
    tid = lax.axis_index("s")

    def run(h_ref, out_ref):
        for j in range(STRIPE // ZCH):
            pltpu.sync_copy(zeros_hbm, acc.at[pl.ds(tid * STRIPE + j * ZCH, ZCH)])
        plsc.subcore_barrier()
        base = tid * RPT

        def body(blk, carry):
            off = base + blk * K
            pltpu.sync_copy(src_rs.at[pl.ds(off, K)], sidx)
            pltpu.sync_copy(dst_rs.at[pl.ds(off, K)], didx)
            descs = [
                pltpu.async_copy(h_ref.at[sidx.at[k]], rows.at[k], sem)
                for k in range(K)
            ]
            for d in descs:
                d.wait()
            for k in range(K):
                pltpu.sync_copy(rows.at[k], acc.at[didx.at[k]], add=True)
            return carry

        lax.fori_loop(0, NBLK, body, 0)
        plsc.subcore_barrier()
        sl = pl.ds(tid * STRIPE, STRIPE)
        pltpu.sync_copy(acc.at[sl], out_ref.at[sl])

    @pl.when(cid == 0)
    def _():
        run(h0, out0)

    @pl.when(cid == 1)
    def _():
        run(h1, out1)


_agg_call = functools.partial(
    pl.kernel,
    out_type=(
        jax.ShapeDtypeStruct((NPAD, 32), jnp.float32),
        jax.ShapeDtypeStruct((NPAD, 32), jnp.float32),
    ),
    mesh=_mesh,
    scratch_types=[
        pltpu.VMEM((K, EPB), jnp.int32),
        pltpu.VMEM((K, EPB), jnp.int32),
        pltpu.VMEM((K, EPB, 32), jnp.float32),
        pltpu.SemaphoreType.DMA,
        pltpu.VMEM_SHARED((NPAD, 32), jnp.float32),
    ],
    compiler_params=_sc_params,
)(_agg_body)


def _down_body(x_ref, w_ref, b_ref, d_ref, h0_ref, h1_ref):
    h = jnp.dot(x_ref[...], w_ref[...],
                preferred_element_type=jnp.float32,
                precision=lax.Precision.HIGHEST) + b_ref[...]
    norm = lax.rsqrt(jnp.maximum(d_ref[:, :1], 1.0))
    h = h * norm
    h0_ref[...] = h[:, :32]
    h1_ref[...] = h[:, 32:]


def _down_call(x, w, b, deg):
    return pl.pallas_call(
        _down_body,
        grid=(GRID,),
        in_specs=[
            pl.BlockSpec((BLK, IN_DIM), lambda i: (i, 0)),
            pl.BlockSpec((IN_DIM, HID), lambda i: (0, 0)),
            pl.BlockSpec((1, HID), lambda i: (0, 0)),
            pl.BlockSpec((BLK, 16), lambda i: (i, 0)),
        ],
        out_specs=(
            pl.BlockSpec((BLK, 32), lambda i: (i, 0)),
            pl.BlockSpec((BLK, 32), lambda i: (i, 0)),
        ),
        out_shape=(
            jax.ShapeDtypeStruct((NPAD, 32), jnp.float32),
            jax.ShapeDtypeStruct((NPAD, 32), jnp.float32),
        ),
    )(x, w, b, deg)


def _up_body(a0_ref, a1_ref, d_ref, wg_ref, bg_ref, wu_ref, bu_ref, o_ref):
    a = jnp.concatenate([a0_ref[...], a1_ref[...]], axis=1)
    norm = lax.rsqrt(jnp.maximum(d_ref[:, :1], 1.0))
    a = a * norm
    g = jnp.dot(a, wg_ref[...],
                preferred_element_type=jnp.float32,
                precision=lax.Precision.HIGHEST) + bg_ref[...]
    o_ref[...] = jnp.dot(g, wu_ref[...],
                         preferred_element_type=jnp.float32,
                         precision=lax.Precision.HIGHEST) + bu_ref[...]


def _up_call(a0, a1, deg, wg, bg, wu, bu):
    return pl.pallas_call(
        _up_body,
        grid=(GRID,),
        in_specs=[
            pl.BlockSpec((BLK, 32), lambda i: (i, 0)),
            pl.BlockSpec((BLK, 32), lambda i: (i, 0)),
            pl.BlockSpec((BLK, 16), lambda i: (i, 0)),
            pl.BlockSpec((HID, HID), lambda i: (0, 0)),
            pl.BlockSpec((1, HID), lambda i: (0, 0)),
            pl.BlockSpec((HID, UP_DIM), lambda i: (0, 0)),
            pl.BlockSpec((1, UP_DIM), lambda i: (0, 0)),
        ],
        out_specs=pl.BlockSpec((BLK, UP_DIM), lambda i: (i, 0)),
        out_shape=jax.ShapeDtypeStruct((NPAD, UP_DIM), jnp.float32),
    )(a0, a1, deg, wg, bg, wu, bu)


def kernel(features, edge_index, W_down, b_down, W_gnn, b_gnn, W_up, b_up):
    src = edge_index[0]
    dst = edge_index[1]
    npad_extra = NPAD - N
    pad_idx = N + (jnp.arange(EPAD - E, dtype=jnp.int32) % npad_extra)
    src_rs = jnp.concatenate([src, pad_idx]).reshape(ROWS, EPB)
    dst_rs = jnp.concatenate([dst, pad_idx]).reshape(ROWS, EPB)

    ones16 = jnp.ones((EPB, 16), jnp.float32)
    zeros16 = jnp.zeros((ZCH, 16), jnp.float32)
    zeros32 = jnp.zeros((ZCH, 32), jnp.float32)

    deg_src, deg_dst = _deg_call(src_rs, dst_rs, ones16, zeros16)

    feats_pad = jnp.pad(features, ((0, npad_extra), (0, 0)))
    h0, h1 = _down_call(feats_pad, W_down, b_down.reshape(1, HID), deg_src)

    agg0, agg1 = _agg_call(src_rs, dst_rs, h0, h1, zeros32)

    out = _up_call(agg0, agg1, deg_dst, W_gnn, b_gnn.reshape(1, HID),
                   W_up, b_up.reshape(1, UP_DIM))
    return out[:N]


# speedup vs baseline: 6.8322x; 1.2154x over previous
import functools

import jax
import jax.numpy as jnp
from jax import lax
from jax.experimental import pallas as pl
from jax.experimental.pallas import tpu as pltpu
from jax.experimental.pallas import tpu_sc as plsc

N = 50000
E = 800000
IN_DIM = 128
HID = 64
UP_DIM = 768

NPAD = 50176
STRIPE = NPAD // 16
ZCH = 784

EPB = 128
RPT = 392
EPAD = 16 * RPT * EPB
ROWS = EPAD // EPB
K = 4
NBLK = RPT // K

BLK = 400
GRID = N // BLK

_mesh = plsc.VectorSubcoreMesh(core_axis_name="c", subcore_axis_name="s")
_sc_params = pltpu.CompilerParams(use_tc_tiling_on_sc=False)


def _deg_body(src_rs, dst_rs, ones_hbm, zeros_hbm, out_src, out_dst,
              idx_v, ones_v, hist):
    cid = lax.axis_index("c")
    tid = lax.axis_index("s")

    def run(eidx, out_ref):
        for j in range(STRIPE // ZCH):
            pltpu.sync_copy(zeros_hbm, hist.at[pl.ds(tid * STRIPE + j * ZCH, ZCH)])
        pltpu.sync_copy(ones_hbm, ones_v)
        plsc.subcore_barrier()
        base = tid * RPT

        def body(blk, carry):
            off = base + blk * K
            pltpu.sync_copy(eidx.at[pl.ds(off, K)], idx_v)
            for k in range(K):
                pltpu.sync_copy(ones_v, hist.at[idx_v.at[k]], add=True)
            return carry

        lax.fori_loop(0, NBLK, body, 0)
        plsc.subcore_barrier()
        sl = pl.ds(tid * STRIPE, STRIPE)
        pltpu.sync_copy(hist.at[sl], out_ref.at[sl])

    @pl.when(cid == 0)
    def _():
        run(src_rs, out_src)

    @pl.when(cid == 1)
    def _():
        run(dst_rs, out_dst)


_deg_call = functools.partial(
    pl.kernel,
    out_type=(
        jax.ShapeDtypeStruct((NPAD, 16), jnp.float32),
        jax.ShapeDtypeStruct((NPAD, 16), jnp.float32),
    ),
    mesh=_mesh,
    scratch_types=[
        pltpu.VMEM((K, EPB), jnp.int32),
        pltpu.VMEM((EPB, 16), jnp.float32),
        pltpu.VMEM_SHARED((NPAD, 16), jnp.float32),
    ],
    compiler_params=_sc_params,
)(_deg_body)


def _agg_body(src_rs, dst_rs, h0, h1, zeros_hbm, out0, out1,
              sidx, didx, rows, sem, acc):
    cid = lax.axis_index("c")
    tid = lax.axis_index("s")

    def run(h_ref, out_ref):
        for j in range(STRIPE // ZCH):
            pltpu.sync_copy(zeros_hbm, acc.at[pl.ds(tid * STRIPE + j * ZCH, ZCH)])
        plsc.subcore_barrier()
        base = tid * RPT

        def body(blk, carry):
            off = base + blk * K
            pltpu.sync_copy(src_rs.at[pl.ds(off, K)], sidx)
            pltpu.sync_copy(dst_rs.at[pl.ds(off, K)], didx)
            descs = [
                pltpu.async_copy(h_ref.at[sidx.at[k]], rows.at[k], sem)
                for k in range(K)
            ]
            for d in descs:
                d.wait()
            for k in range(K):
                pltpu.sync_copy(rows.at[k], acc.at[didx.at[k]], add=True)
            return carry

        lax.fori_loop(0, NBLK, body, 0)
        plsc.subcore_barrier()
        sl = pl.ds(tid * STRIPE, STRIPE)
        pltpu.sync_copy(acc.at[sl], out_ref.at[sl])

    @pl.when(cid == 0)
    def _():
        run(h0, out0)

    @pl.when(cid == 1)
    def _():
        run(h1, out1)


_agg_call = functools.partial(
    pl.kernel,
    out_type=(
        jax.ShapeDtypeStruct((NPAD, 32), jnp.float32),
        jax.ShapeDtypeStruct((NPAD, 32), jnp.float32),
    ),
    mesh=_mesh,
    scratch_types=[
        pltpu.VMEM((K, EPB), jnp.int32),
        pltpu.VMEM((K, EPB), jnp.int32),
        pltpu.VMEM((K, EPB, 32), jnp.float32),
        pltpu.SemaphoreType.DMA,
        pltpu.VMEM_SHARED((NPAD, 32), jnp.float32),
    ],
    compiler_params=_sc_params,
)(_agg_body)


def _down_body(x_ref, w_ref, b_ref, d_ref, h0_ref, h1_ref):
    h = jnp.dot(x_ref[...], w_ref[...],
                preferred_element_type=jnp.float32,
                precision=lax.Precision.HIGHEST) + b_ref[...]
    norm = lax.rsqrt(jnp.maximum(d_ref[:, :1], 1.0))
    h = h * norm
    h0_ref[...] = h[:, :32]
    h1_ref[...] = h[:, 32:]


def _down_call(x, w, b, deg):
    return pl.pallas_call(
        _down_body,
        grid=(GRID,),
        in_specs=[
            pl.BlockSpec((BLK, IN_DIM), lambda i: (i, 0)),
            pl.BlockSpec((IN_DIM, HID), lambda i: (0, 0)),
            pl.BlockSpec((1, HID), lambda i: (0, 0)),
            pl.BlockSpec((BLK, 16), lambda i: (i, 0)),
        ],
        out_specs=(
            pl.BlockSpec((BLK, 32), lambda i: (i, 0)),
            pl.BlockSpec((BLK, 32), lambda i: (i, 0)),
        ),
        out_shape=(
            jax.ShapeDtypeStruct((N, 32), jnp.float32),
            jax.ShapeDtypeStruct((N, 32), jnp.float32),
        ),
    )(x, w, b, deg)


def _fuse_body(wg_ref, bg_ref, wu_ref, bu_ref, wf_ref, bf_ref):
    wf_ref[...] = jnp.dot(wg_ref[...], wu_ref[...],
                          preferred_element_type=jnp.float32,
                          precision=lax.Precision.HIGHEST)
    bf_ref[...] = jnp.dot(bg_ref[...], wu_ref[...],
                          preferred_element_type=jnp.float32,
                          precision=lax.Precision.HIGHEST) + bu_ref[...]


def _fuse_call(wg, bg, wu, bu):
    return pl.pallas_call(
        _fuse_body,
        out_shape=(
            jax.ShapeDtypeStruct((HID, UP_DIM), jnp.float32),
            jax.ShapeDtypeStruct((1, UP_DIM), jnp.float32),
        ),
    )(wg, bg, wu, bu)


def _up_body(a0_ref, a1_ref, d_ref, wf_ref, bf_ref, o_ref):
    a = jnp.concatenate([a0_ref[...], a1_ref[...]], axis=1)
    norm = lax.rsqrt(jnp.maximum(d_ref[:, :1], 1.0))
    a = a * norm
    o_ref[...] = jnp.dot(a, wf_ref[...],
                         preferred_element_type=jnp.float32,
                         precision=lax.Precision.DEFAULT) + bf_ref[...]


def _up_call(a0, a1, deg, wf, bf):
    return pl.pallas_call(
        _up_body,
        grid=(GRID,),
        in_specs=[
            pl.BlockSpec((BLK, 32), lambda i: (i, 0)),
            pl.BlockSpec((BLK, 32), lambda i: (i, 0)),
            pl.BlockSpec((BLK, 16), lambda i: (i, 0)),
            pl.BlockSpec((HID, UP_DIM), lambda i: (0, 0)),
            pl.BlockSpec((1, UP_DIM), lambda i: (0, 0)),
        ],
        out_specs=pl.BlockSpec((BLK, UP_DIM), lambda i: (i, 0)),
        out_shape=jax.ShapeDtypeStruct((N, UP_DIM), jnp.float32),
    )(a0, a1, deg, wf, bf)


def kernel(features, edge_index, W_down, b_down, W_gnn, b_gnn, W_up, b_up):
    src = edge_index[0]
    dst = edge_index[1]
    npad_extra = NPAD - N
    pad_n = EPAD - E
    pad_trash = N + (jnp.arange(pad_n, dtype=jnp.int32) % npad_extra)
    pad_low = jnp.arange(pad_n, dtype=jnp.int32) % EPB
    src_deg_rs = jnp.concatenate([src, pad_trash]).reshape(ROWS, EPB)
    src_gat_rs = jnp.concatenate([src, pad_low]).reshape(ROWS, EPB)
    dst_rs = jnp.concatenate([dst, pad_trash]).reshape(ROWS, EPB)

    ones16 = jnp.ones((EPB, 16), jnp.float32)
    zeros16 = jnp.zeros((ZCH, 16), jnp.float32)
    zeros32 = jnp.zeros((ZCH, 32), jnp.float32)

    deg_src, deg_dst = _deg_call(src_deg_rs, dst_rs, ones16, zeros16)

    h0, h1 = _down_call(features, W_down, b_down.reshape(1, HID), deg_src)

    agg0, agg1 = _agg_call(src_gat_rs, dst_rs, h0, h1, zeros32)

    wf, bf = _fuse_call(W_gnn, b_gnn.reshape(1, HID), W_up,
                        b_up.reshape(1, UP_DIM))
    return _up_call(agg0, agg1, deg_dst, wf, bf)


# speedup vs baseline: 7.4540x; 1.0910x over previous
import functools

import jax
import jax.numpy as jnp
from jax import lax
from jax.experimental import pallas as pl
from jax.experimental.pallas import tpu as pltpu
from jax.experimental.pallas import tpu_sc as plsc

N = 50000
E = 800000
IN_DIM = 128
HID = 64
UP_DIM = 768

NPAD = 50176
STRIPE = NPAD // 16
ZCH = 784

EPB = 128
RPT = 396
EPAD = 16 * RPT * EPB
ROWS = EPAD // EPB
K = 3
NBLK = RPT // K

BLK = 400
GRID = N // BLK

_mesh = plsc.VectorSubcoreMesh(core_axis_name="c", subcore_axis_name="s")
_sc_params = pltpu.CompilerParams(use_tc_tiling_on_sc=False)


def _deg_body(src_rs, dst_rs, ones_hbm, zeros_hbm, out_src, out_dst,
              idx_v, ones_v, hist):
    cid = lax.axis_index("c")
    tid = lax.axis_index("s")

    def run(eidx, out_ref):
        for j in range(STRIPE // ZCH):
            pltpu.sync_copy(zeros_hbm, hist.at[pl.ds(tid * STRIPE + j * ZCH, ZCH)])
        pltpu.sync_copy(ones_hbm, ones_v)
        plsc.subcore_barrier()
        base = tid * RPT

        def body(blk, carry):
            off = base + blk * K
            pltpu.sync_copy(eidx.at[pl.ds(off, K)], idx_v)
            for k in range(K):
                pltpu.sync_copy(ones_v, hist.at[idx_v.at[k]], add=True)
            return carry

        lax.fori_loop(0, NBLK, body, 0)
        plsc.subcore_barrier()
        sl = pl.ds(tid * STRIPE, STRIPE)
        pltpu.sync_copy(hist.at[sl], out_ref.at[sl])

    @pl.when(cid == 0)
    def _():
        run(src_rs, out_src)

    @pl.when(cid == 1)
    def _():
        run(dst_rs, out_dst)


_deg_call = functools.partial(
    pl.kernel,
    out_type=(
        jax.ShapeDtypeStruct((NPAD, 16), jnp.float32),
        jax.ShapeDtypeStruct((NPAD, 16), jnp.float32),
    ),
    mesh=_mesh,
    scratch_types=[
        pltpu.VMEM((K, EPB), jnp.int32),
        pltpu.VMEM((EPB, 16), jnp.float32),
        pltpu.VMEM_SHARED((NPAD, 16), jnp.float32),
    ],
    compiler_params=_sc_params,
)(_deg_body)


def _agg_body(src_rs, dst_rs, h0, h1, zeros_hbm, out0, out1,
              sidxA, didxA, sidxB, didxB, rowsA, rowsB, semA, semB, acc):
    cid = lax.axis_index("c")
    tid = lax.axis_index("s")

    def run(h_ref, out_ref):
        for j in range(STRIPE // ZCH):
            pltpu.sync_copy(zeros_hbm, acc.at[pl.ds(tid * STRIPE + j * ZCH, ZCH)])
        plsc.subcore_barrier()
        base = tid * RPT

        def issue(b, sidx, didx, rows, sem):
            off = base + b * K
            pltpu.sync_copy(src_rs.at[pl.ds(off, K)], sidx)
            pltpu.sync_copy(dst_rs.at[pl.ds(off, K)], didx)
            for k in range(K):
                pltpu.async_copy(h_ref.at[sidx.at[k]], rows.at[k], sem)

        def drain_scatter(sidx, didx, rows, sem):
            for k in range(K):
                pltpu.make_async_copy(h_ref.at[sidx.at[k]], rows.at[k], sem).wait()
            for k in range(K):
                pltpu.sync_copy(rows.at[k], acc.at[didx.at[k]], add=True)

        issue(0, sidxA, didxA, rowsA, semA)

        def body(sb, carry):
            b0 = 2 * sb
            issue(b0 + 1, sidxB, didxB, rowsB, semB)
            drain_scatter(sidxA, didxA, rowsA, semA)

            @pl.when(sb + 1 < NBLK // 2)
            def _():
                issue(b0 + 2, sidxA, didxA, rowsA, semA)

            drain_scatter(sidxB, didxB, rowsB, semB)
            return carry

        lax.fori_loop(0, NBLK // 2, body, 0)
        plsc.subcore_barrier()
        sl = pl.ds(tid * STRIPE, STRIPE)
        pltpu.sync_copy(acc.at[sl], out_ref.at[sl])

    @pl.when(cid == 0)
    def _():
        run(h0, out0)

    @pl.when(cid == 1)
    def _():
        run(h1, out1)


_agg_call = functools.partial(
    pl.kernel,
    out_type=(
        jax.ShapeDtypeStruct((NPAD, 32), jnp.float32),
        jax.ShapeDtypeStruct((NPAD, 32), jnp.float32),
    ),
    mesh=_mesh,
    scratch_types=[
        pltpu.VMEM((K, EPB), jnp.int32),
        pltpu.VMEM((K, EPB), jnp.int32),
        pltpu.VMEM((K, EPB), jnp.int32),
        pltpu.VMEM((K, EPB), jnp.int32),
        pltpu.VMEM((K, EPB, 32), jnp.float32),
        pltpu.VMEM((K, EPB, 32), jnp.float32),
        pltpu.SemaphoreType.DMA,
        pltpu.SemaphoreType.DMA,
        pltpu.VMEM_SHARED((NPAD, 32), jnp.float32),
    ],
    compiler_params=_sc_params,
)(_agg_body)


def _down_body(x_ref, w_ref, b_ref, d_ref, h0_ref, h1_ref):
    h = jnp.dot(x_ref[...], w_ref[...],
                preferred_element_type=jnp.float32,
                precision=lax.Precision.HIGHEST) + b_ref[...]
    norm = lax.rsqrt(jnp.maximum(d_ref[:, :1], 1.0))
    h = h * norm
    h0_ref[...] = h[:, :32]
    h1_ref[...] = h[:, 32:]


def _down_call(x, w, b, deg):
    return pl.pallas_call(
        _down_body,
        grid=(GRID,),
        in_specs=[
            pl.BlockSpec((BLK, IN_DIM), lambda i: (i, 0)),
            pl.BlockSpec((IN_DIM, HID), lambda i: (0, 0)),
            pl.BlockSpec((1, HID), lambda i: (0, 0)),
            pl.BlockSpec((BLK, 16), lambda i: (i, 0)),
        ],
        out_specs=(
            pl.BlockSpec((BLK, 32), lambda i: (i, 0)),
            pl.BlockSpec((BLK, 32), lambda i: (i, 0)),
        ),
        out_shape=(
            jax.ShapeDtypeStruct((N, 32), jnp.float32),
            jax.ShapeDtypeStruct((N, 32), jnp.float32),
        ),
    )(x, w, b, deg)


def _fuse_body(wg_ref, bg_ref, wu_ref, bu_ref, wf_ref, bf_ref):
    wf_ref[...] = jnp.dot(wg_ref[...], wu_ref[...],
                          preferred_element_type=jnp.float32,
                          precision=lax.Precision.HIGHEST)
    bf_ref[...] = jnp.dot(bg_ref[...], wu_ref[...],
                          preferred_element_type=jnp.float32,
                          precision=lax.Precision.HIGHEST) + bu_ref[...]


def _fuse_call(wg, bg, wu, bu):
    return pl.pallas_call(
        _fuse_body,
        out_shape=(
            jax.ShapeDtypeStruct((HID, UP_DIM), jnp.float32),
            jax.ShapeDtypeStruct((1, UP_DIM), jnp.float32),
        ),
    )(wg, bg, wu, bu)


def _up_body(a0_ref, a1_ref, d_ref, wf_ref, bf_ref, o_ref):
    a = jnp.concatenate([a0_ref[...], a1_ref[...]], axis=1)
    norm = lax.rsqrt(jnp.maximum(d_ref[:, :1], 1.0))
    a = a * norm
    o_ref[...] = jnp.dot(a, wf_ref[...],
                         preferred_element_type=jnp.float32,
                         precision=lax.Precision.DEFAULT) + bf_ref[...]


def _up_call(a0, a1, deg, wf, bf):
    return pl.pallas_call(
        _up_body,
        grid=(GRID,),
        in_specs=[
            pl.BlockSpec((BLK, 32), lambda i: (i, 0)),
            pl.BlockSpec((BLK, 32), lambda i: (i, 0)),
            pl.BlockSpec((BLK, 16), lambda i: (i, 0)),
            pl.BlockSpec((HID, UP_DIM), lambda i: (0, 0)),
            pl.BlockSpec((1, UP_DIM), lambda i: (0, 0)),
        ],
        out_specs=pl.BlockSpec((BLK, UP_DIM), lambda i: (i, 0)),
        out_shape=jax.ShapeDtypeStruct((N, UP_DIM), jnp.float32),
    )(a0, a1, deg, wf, bf)


def kernel(features, edge_index, W_down, b_down, W_gnn, b_gnn, W_up, b_up):
    src = edge_index[0]
    dst = edge_index[1]
    npad_extra = NPAD - N
    pad_n = EPAD - E
    pad_trash = N + (jnp.arange(pad_n, dtype=jnp.int32) % npad_extra)
    pad_low = jnp.arange(pad_n, dtype=jnp.int32) % EPB
    src_deg_rs = jnp.concatenate([src, pad_trash]).reshape(ROWS, EPB)
    src_gat_rs = jnp.concatenate([src, pad_low]).reshape(ROWS, EPB)
    dst_rs = jnp.concatenate([dst, pad_trash]).reshape(ROWS, EPB)

    ones16 = jnp.ones((EPB, 16), jnp.float32)
    zeros16 = jnp.zeros((ZCH, 16), jnp.float32)
    zeros32 = jnp.zeros((ZCH, 32), jnp.float32)

    deg_src, deg_dst = _deg_call(src_deg_rs, dst_rs, ones16, zeros16)

    h0, h1 = _down_call(features, W_down, b_down.reshape(1, HID), deg_src)

    agg0, agg1 = _agg_call(src_gat_rs, dst_rs, h0, h1, zeros32)

    wf, bf = _fuse_call(W_gnn, b_gnn.reshape(1, HID), W_up,
                        b_up.reshape(1, UP_DIM))
    return _up_call(agg0, agg1, deg_dst, wf, bf)


# speedup vs baseline: 8.1784x; 1.0972x over previous
import functools

import jax
import jax.numpy as jnp
from jax import lax
from jax.experimental import pallas as pl
from jax.experimental.pallas import tpu as pltpu
from jax.experimental.pallas import tpu_sc as plsc

N = 50000
E = 800000
IN_DIM = 128
HID = 64
UP_DIM = 768

NPAD = 50176
STRIPE = NPAD // 16
ZCH = 784

EPB = 128
RPT = 396
EPAD = 16 * RPT * EPB
ROWS = EPAD // EPB
K = 3
NBLK = RPT // K

BLK = 400
GRID = N // BLK

_mesh = plsc.VectorSubcoreMesh(core_axis_name="c", subcore_axis_name="s")
_sc_params = pltpu.CompilerParams(use_tc_tiling_on_sc=False)


def _deg_body(src_rs, dst_rs, ones_hbm, zeros_hbm, out_src, out_dst,
              idx_v, ones_v, hist):
    cid = lax.axis_index("c")
    tid = lax.axis_index("s")

    def run(eidx, out_ref):
        for j in range(STRIPE // ZCH):
            pltpu.sync_copy(zeros_hbm, hist.at[pl.ds(tid * STRIPE + j * ZCH, ZCH)])
        pltpu.sync_copy(ones_hbm, ones_v)
        plsc.subcore_barrier()
        base = tid * RPT

        def body(blk, carry):
            off = base + blk * K
            pltpu.sync_copy(eidx.at[pl.ds(off, K)], idx_v)
            for k in range(K):
                pltpu.sync_copy(ones_v, hist.at[idx_v.at[k]], add=True)
            return carry

        lax.fori_loop(0, NBLK, body, 0)
        plsc.subcore_barrier()
        sl = pl.ds(tid * STRIPE, STRIPE)
        pltpu.sync_copy(hist.at[sl], out_ref.at[sl])

    @pl.when(cid == 0)
    def _():
        run(src_rs, out_src)

    @pl.when(cid == 1)
    def _():
        run(dst_rs, out_dst)


_deg_call = functools.partial(
    pl.kernel,
    out_type=(
        jax.ShapeDtypeStruct((NPAD, 16), jnp.float32),
        jax.ShapeDtypeStruct((NPAD, 16), jnp.float32),
    ),
    mesh=_mesh,
    scratch_types=[
        pltpu.VMEM((K, EPB), jnp.int32),
        pltpu.VMEM((EPB, 16), jnp.float32),
        pltpu.VMEM_SHARED((NPAD, 16), jnp.float32),
    ],
    compiler_params=_sc_params,
)(_deg_body)


def _agg_body(sd_rs, h0, h1, zeros_hbm, out0, out1,
              idxA, idxB, rowsA, rowsB, semA, semB, semSA, semSB, acc):
    cid = lax.axis_index("c")
    tid = lax.axis_index("s")
    NP = NBLK // 2

    def run(h_ref, out_ref):
        for j in range(STRIPE // ZCH):
            pltpu.sync_copy(zeros_hbm, acc.at[pl.ds(tid * STRIPE + j * ZCH, ZCH)])
        plsc.subcore_barrier()
        base = tid * RPT

        def issue(b, idx, rows, sem):
            pltpu.sync_copy(sd_rs.at[pl.ds(base + b * K, K)], idx)
            for k in range(K):
                pltpu.async_copy(h_ref.at[idx.at[k, 0]], rows.at[k], sem)

        def wait_g(idx, rows, sem):
            for k in range(K):
                pltpu.make_async_copy(h_ref.at[idx.at[k, 0]], rows.at[k], sem).wait()

        def scatter(idx, rows, semS):
            for k in range(K):
                pltpu.async_copy(rows.at[k], acc.at[idx.at[k, 1]], semS, add=True)

        def wait_s(idx, rows, semS):
            for k in range(K):
                pltpu.make_async_copy(rows.at[k], acc.at[idx.at[k, 1]], semS).wait()

        issue(0, idxA, rowsA, semA)

        def body(sb, carry):
            b0 = 2 * sb

            @pl.when(sb > 0)
            def _():
                wait_s(idxB, rowsB, semSB)

            issue(b0 + 1, idxB, rowsB, semB)
            wait_g(idxA, rowsA, semA)
            scatter(idxA, rowsA, semSA)

            @pl.when(sb + 1 < NP)
            def _():
                wait_s(idxA, rowsA, semSA)
                issue(b0 + 2, idxA, rowsA, semA)

            wait_g(idxB, rowsB, semB)
            scatter(idxB, rowsB, semSB)
            return carry

        lax.fori_loop(0, NP, body, 0)
        wait_s(idxA, rowsA, semSA)
        wait_s(idxB, rowsB, semSB)
        plsc.subcore_barrier()
        sl = pl.ds(tid * STRIPE, STRIPE)
        pltpu.sync_copy(acc.at[sl], out_ref.at[sl])

    @pl.when(cid == 0)
    def _():
        run(h0, out0)

    @pl.when(cid == 1)
    def _():
        run(h1, out1)


_agg_call = functools.partial(
    pl.kernel,
    out_type=(
        jax.ShapeDtypeStruct((NPAD, 32), jnp.float32),
        jax.ShapeDtypeStruct((NPAD, 32), jnp.float32),
    ),
    mesh=_mesh,
    scratch_types=[
        pltpu.VMEM((K, 2, EPB), jnp.int32),
        pltpu.VMEM((K, 2, EPB), jnp.int32),
        pltpu.VMEM((K, EPB, 32), jnp.float32),
        pltpu.VMEM((K, EPB, 32), jnp.float32),
        pltpu.SemaphoreType.DMA,
        pltpu.SemaphoreType.DMA,
        pltpu.SemaphoreType.DMA,
        pltpu.SemaphoreType.DMA,
        pltpu.VMEM_SHARED((NPAD, 32), jnp.float32),
    ],
    compiler_params=_sc_params,
)(_agg_body)


def _down_body(x_ref, w_ref, b_ref, d_ref, h0_ref, h1_ref):
    h = jnp.dot(x_ref[...], w_ref[...],
                preferred_element_type=jnp.float32,
                precision=lax.Precision.HIGHEST) + b_ref[...]
    norm = lax.rsqrt(jnp.maximum(d_ref[:, :1], 1.0))
    h = h * norm
    h0_ref[...] = h[:, :32]
    h1_ref[...] = h[:, 32:]


def _down_call(x, w, b, deg):
    return pl.pallas_call(
        _down_body,
        grid=(GRID,),
        in_specs=[
            pl.BlockSpec((BLK, IN_DIM), lambda i: (i, 0)),
            pl.BlockSpec((IN_DIM, HID), lambda i: (0, 0)),
            pl.BlockSpec((1, HID), lambda i: (0, 0)),
            pl.BlockSpec((BLK, 16), lambda i: (i, 0)),
        ],
        out_specs=(
            pl.BlockSpec((BLK, 32), lambda i: (i, 0)),
            pl.BlockSpec((BLK, 32), lambda i: (i, 0)),
        ),
        out_shape=(
            jax.ShapeDtypeStruct((N, 32), jnp.float32),
            jax.ShapeDtypeStruct((N, 32), jnp.float32),
        ),
    )(x, w, b, deg)


def _fuse_body(wg_ref, bg_ref, wu_ref, bu_ref, wf_ref, bf_ref):
    wf_ref[...] = jnp.dot(wg_ref[...], wu_ref[...],
                          preferred_element_type=jnp.float32,
                          precision=lax.Precision.HIGHEST)
    bf_ref[...] = jnp.dot(bg_ref[...], wu_ref[...],
                          preferred_element_type=jnp.float32,
                          precision=lax.Precision.HIGHEST) + bu_ref[...]


def _fuse_call(wg, bg, wu, bu):
    return pl.pallas_call(
        _fuse_body,
        out_shape=(
            jax.ShapeDtypeStruct((HID, UP_DIM), jnp.float32),
            jax.ShapeDtypeStruct((1, UP_DIM), jnp.float32),
        ),
    )(wg, bg, wu, bu)


def _up_body(a0_ref, a1_ref, d_ref, wf_ref, bf_ref, o_ref):
    a = jnp.concatenate([a0_ref[...], a1_ref[...]], axis=1)
    norm = lax.rsqrt(jnp.maximum(d_ref[:, :1], 1.0))
    a = a * norm
    o_ref[...] = jnp.dot(a, wf_ref[...],
                         preferred_element_type=jnp.float32,
                         precision=lax.Precision.DEFAULT) + bf_ref[...]


def _up_call(a0, a1, deg, wf, bf):
    return pl.pallas_call(
        _up_body,
        grid=(GRID,),
        in_specs=[
            pl.BlockSpec((BLK, 32), lambda i: (i, 0)),
            pl.BlockSpec((BLK, 32), lambda i: (i, 0)),
            pl.BlockSpec((BLK, 16), lambda i: (i, 0)),
            pl.BlockSpec((HID, UP_DIM), lambda i: (0, 0)),
            pl.BlockSpec((1, UP_DIM), lambda i: (0, 0)),
        ],
        out_specs=pl.BlockSpec((BLK, UP_DIM), lambda i: (i, 0)),
        out_shape=jax.ShapeDtypeStruct((N, UP_DIM), jnp.float32),
    )(a0, a1, deg, wf, bf)


def kernel(features, edge_index, W_down, b_down, W_gnn, b_gnn, W_up, b_up):
    src = edge_index[0]
    dst = edge_index[1]
    npad_extra = NPAD - N
    pad_n = EPAD - E
    pad_trash = N + (jnp.arange(pad_n, dtype=jnp.int32) % npad_extra)
    pad_low = jnp.arange(pad_n, dtype=jnp.int32) % EPB
    src_deg_rs = jnp.concatenate([src, pad_trash]).reshape(ROWS, EPB)
    src_gat_rs = jnp.concatenate([src, pad_low]).reshape(ROWS, EPB)
    dst_rs = jnp.concatenate([dst, pad_trash]).reshape(ROWS, EPB)
    sd_rs = jnp.stack([src_gat_rs, dst_rs], axis=1)

    ones16 = jnp.ones((EPB, 16), jnp.float32)
    zeros16 = jnp.zeros((ZCH, 16), jnp.float32)
    zeros32 = jnp.zeros((ZCH, 32), jnp.float32)

    deg_src, deg_dst = _deg_call(src_deg_rs, dst_rs, ones16, zeros16)

    h0, h1 = _down_call(features, W_down, b_down.reshape(1, HID), deg_src)

    agg0, agg1 = _agg_call(sd_rs, h0, h1, zeros32)

    wf, bf = _fuse_call(W_gnn, b_gnn.reshape(1, HID), W_up,
                        b_up.reshape(1, UP_DIM))
    return _up_call(agg0, agg1, deg_dst, wf, bf)


# speedup vs baseline: 8.8386x; 1.0807x over previous
import functools

import jax
import jax.numpy as jnp
from jax import lax
from jax.experimental import pallas as pl
from jax.experimental.pallas import tpu as pltpu
from jax.experimental.pallas import tpu_sc as plsc

N = 50000
E = 800000
IN_DIM = 128
HID = 64
UP_DIM = 768

NPAD = 50176
STRIPE = NPAD // 16
ZCH = 784

EPB = 128
RPT = 396
EPAD = 16 * RPT * EPB
ROWS = EPAD // EPB
K = 3
NBLK = RPT // K

BLK = 400
GRID = N // BLK

_mesh = plsc.VectorSubcoreMesh(core_axis_name="c", subcore_axis_name="s")
_sc_params = pltpu.CompilerParams(use_tc_tiling_on_sc=False)


def _deg_body(src_rs, dst_rs, ones_hbm, zeros_hbm, out_src, out_dst,
              idxA, idxB, ones_v, semSA, semSB, hist):
    cid = lax.axis_index("c")
    tid = lax.axis_index("s")
    NP = NBLK // 2

    def run(eidx, out_ref):
        for j in range(STRIPE // ZCH):
            pltpu.sync_copy(zeros_hbm, hist.at[pl.ds(tid * STRIPE + j * ZCH, ZCH)])
        pltpu.sync_copy(ones_hbm, ones_v)
        plsc.subcore_barrier()
        base = tid * RPT

        def issue(b, idx, semS):
            pltpu.sync_copy(eidx.at[pl.ds(base + b * K, K)], idx)
            for k in range(K):
                pltpu.async_copy(ones_v, hist.at[idx.at[k]], semS, add=True)

        def wait_s(idx, semS):
            for k in range(K):
                pltpu.make_async_copy(ones_v, hist.at[idx.at[k]], semS).wait()

        issue(0, idxA, semSA)

        def body(sb, carry):
            b0 = 2 * sb

            @pl.when(sb > 0)
            def _():
                wait_s(idxB, semSB)

            issue(b0 + 1, idxB, semSB)

            @pl.when(sb + 1 < NP)
            def _():
                wait_s(idxA, semSA)
                issue(b0 + 2, idxA, semSA)

            return carry

        lax.fori_loop(0, NP, body, 0)
        wait_s(idxA, semSA)
        wait_s(idxB, semSB)
        plsc.subcore_barrier()
        sl = pl.ds(tid * STRIPE, STRIPE)
        pltpu.sync_copy(hist.at[sl], out_ref.at[sl])

    @pl.when(cid == 0)
    def _():
        run(src_rs, out_src)

    @pl.when(cid == 1)
    def _():
        run(dst_rs, out_dst)


_deg_call = functools.partial(
    pl.kernel,
    out_type=(
        jax.ShapeDtypeStruct((NPAD, 16), jnp.float32),
        jax.ShapeDtypeStruct((NPAD, 16), jnp.float32),
    ),
    mesh=_mesh,
    scratch_types=[
        pltpu.VMEM((K, EPB), jnp.int32),
        pltpu.VMEM((K, EPB), jnp.int32),
        pltpu.VMEM((EPB, 16), jnp.float32),
        pltpu.SemaphoreType.DMA,
        pltpu.SemaphoreType.DMA,
        pltpu.VMEM_SHARED((NPAD, 16), jnp.float32),
    ],
    compiler_params=_sc_params,
)(_deg_body)


def _agg_body(sd_rs, h0, h1, zeros_hbm, out0, out1,
              idxA, idxB, rowsA, rowsB, semA, semB, semSA, semSB, acc):
    cid = lax.axis_index("c")
    tid = lax.axis_index("s")
    NP = NBLK // 2

    def run(h_ref, out_ref):
        for j in range(STRIPE // ZCH):
            pltpu.sync_copy(zeros_hbm, acc.at[pl.ds(tid * STRIPE + j * ZCH, ZCH)])
        plsc.subcore_barrier()
        base = tid * RPT

        def issue(b, idx, rows, sem):
            pltpu.sync_copy(sd_rs.at[pl.ds(base + b * K, K)], idx)
            for k in range(K):
                pltpu.async_copy(h_ref.at[idx.at[k, 0]], rows.at[k], sem)

        def wait_g(idx, rows, sem):
            for k in range(K):
                pltpu.make_async_copy(h_ref.at[idx.at[k, 0]], rows.at[k], sem).wait()

        def scatter(idx, rows, semS):
            for k in range(K):
                pltpu.async_copy(rows.at[k], acc.at[idx.at[k, 1]], semS, add=True)

        def wait_s(idx, rows, semS):
            for k in range(K):
                pltpu.make_async_copy(rows.at[k], acc.at[idx.at[k, 1]], semS).wait()

        issue(0, idxA, rowsA, semA)

        def body(sb, carry):
            b0 = 2 * sb

            @pl.when(sb > 0)
            def _():
                wait_s(idxB, rowsB, semSB)

            issue(b0 + 1, idxB, rowsB, semB)
            wait_g(idxA, rowsA, semA)
            scatter(idxA, rowsA, semSA)

            @pl.when(sb + 1 < NP)
            def _():
                wait_s(idxA, rowsA, semSA)
                issue(b0 + 2, idxA, rowsA, semA)

            wait_g(idxB, rowsB, semB)
            scatter(idxB, rowsB, semSB)
            return carry

        lax.fori_loop(0, NP, body, 0)
        wait_s(idxA, rowsA, semSA)
        wait_s(idxB, rowsB, semSB)
        plsc.subcore_barrier()
        sl = pl.ds(tid * STRIPE, STRIPE)
        pltpu.sync_copy(acc.at[sl], out_ref.at[sl])

    @pl.when(cid == 0)
    def _():
        run(h0, out0)

    @pl.when(cid == 1)
    def _():
        run(h1, out1)


_agg_call = functools.partial(
    pl.kernel,
    out_type=(
        jax.ShapeDtypeStruct((NPAD, 32), jnp.float32),
        jax.ShapeDtypeStruct((NPAD, 32), jnp.float32),
    ),
    mesh=_mesh,
    scratch_types=[
        pltpu.VMEM((K, 2, EPB), jnp.int32),
        pltpu.VMEM((K, 2, EPB), jnp.int32),
        pltpu.VMEM((K, EPB, 32), jnp.float32),
        pltpu.VMEM((K, EPB, 32), jnp.float32),
        pltpu.SemaphoreType.DMA,
        pltpu.SemaphoreType.DMA,
        pltpu.SemaphoreType.DMA,
        pltpu.SemaphoreType.DMA,
        pltpu.VMEM_SHARED((NPAD, 32), jnp.float32),
    ],
    compiler_params=_sc_params,
)(_agg_body)


def _down_body(x_ref, w_ref, b_ref, d_ref, h0_ref, h1_ref):
    h = jnp.dot(x_ref[...], w_ref[...],
                preferred_element_type=jnp.float32,
                precision=lax.Precision.HIGHEST) + b_ref[...]
    norm = lax.rsqrt(jnp.maximum(d_ref[:, :1], 1.0))
    h = h * norm
    h0_ref[...] = h[:, :32]
    h1_ref[...] = h[:, 32:]


def _down_call(x, w, b, deg):
    return pl.pallas_call(
        _down_body,
        grid=(GRID,),
        in_specs=[
            pl.BlockSpec((BLK, IN_DIM), lambda i: (i, 0)),
            pl.BlockSpec((IN_DIM, HID), lambda i: (0, 0)),
            pl.BlockSpec((1, HID), lambda i: (0, 0)),
            pl.BlockSpec((BLK, 16), lambda i: (i, 0)),
        ],
        out_specs=(
            pl.BlockSpec((BLK, 32), lambda i: (i, 0)),
            pl.BlockSpec((BLK, 32), lambda i: (i, 0)),
        ),
        out_shape=(
            jax.ShapeDtypeStruct((N, 32), jnp.float32),
            jax.ShapeDtypeStruct((N, 32), jnp.float32),
        ),
    )(x, w, b, deg)


def _fuse_body(wg_ref, bg_ref, wu_ref, bu_ref, wf_ref, bf_ref):
    wf_ref[...] = jnp.dot(wg_ref[...], wu_ref[...],
                          preferred_element_type=jnp.float32,
                          precision=lax.Precision.HIGHEST)
    bf_ref[...] = jnp.dot(bg_ref[...], wu_ref[...],
                          preferred_element_type=jnp.float32,
                          precision=lax.Precision.HIGHEST) + bu_ref[...]


def _fuse_call(wg, bg, wu, bu):
    return pl.pallas_call(
        _fuse_body,
        out_shape=(
            jax.ShapeDtypeStruct((HID, UP_DIM), jnp.float32),
            jax.ShapeDtypeStruct((1, UP_DIM), jnp.float32),
        ),
    )(wg, bg, wu, bu)


def _up_body(a0_ref, a1_ref, d_ref, wf_ref, bf_ref, o_ref):
    a = jnp.concatenate([a0_ref[...], a1_ref[...]], axis=1)
    norm = lax.rsqrt(jnp.maximum(d_ref[:, :1], 1.0))
    a = a * norm
    o_ref[...] = jnp.dot(a, wf_ref[...],
                         preferred_element_type=jnp.float32,
                         precision=lax.Precision.DEFAULT) + bf_ref[...]


def _up_call(a0, a1, deg, wf, bf):
    return pl.pallas_call(
        _up_body,
        grid=(GRID,),
        in_specs=[
            pl.BlockSpec((BLK, 32), lambda i: (i, 0)),
            pl.BlockSpec((BLK, 32), lambda i: (i, 0)),
            pl.BlockSpec((BLK, 16), lambda i: (i, 0)),
            pl.BlockSpec((HID, UP_DIM), lambda i: (0, 0)),
            pl.BlockSpec((1, UP_DIM), lambda i: (0, 0)),
        ],
        out_specs=pl.BlockSpec((BLK, UP_DIM), lambda i: (i, 0)),
        out_shape=jax.ShapeDtypeStruct((N, UP_DIM), jnp.float32),
    )(a0, a1, deg, wf, bf)


def kernel(features, edge_index, W_down, b_down, W_gnn, b_gnn, W_up, b_up):
    src = edge_index[0]
    dst = edge_index[1]
    npad_extra = NPAD - N
    pad_n = EPAD - E
    pad_trash = N + (jnp.arange(pad_n, dtype=jnp.int32) % npad_extra)
    pad_low = jnp.arange(pad_n, dtype=jnp.int32) % EPB
    src_deg_rs = jnp.concatenate([src, pad_trash]).reshape(ROWS, EPB)
    src_gat_rs = jnp.concatenate([src, pad_low]).reshape(ROWS, EPB)
    dst_rs = jnp.concatenate([dst, pad_trash]).reshape(ROWS, EPB)
    sd_rs = jnp.stack([src_gat_rs, dst_rs], axis=1)

    ones16 = jnp.ones((EPB, 16), jnp.float32)
    zeros16 = jnp.zeros((ZCH, 16), jnp.float32)
    zeros32 = jnp.zeros((ZCH, 32), jnp.float32)

    deg_src, deg_dst = _deg_call(src_deg_rs, dst_rs, ones16, zeros16)

    h0, h1 = _down_call(features, W_down, b_down.reshape(1, HID), deg_src)

    agg0, agg1 = _agg_call(sd_rs, h0, h1, zeros32)

    wf, bf = _fuse_call(W_gnn, b_gnn.reshape(1, HID), W_up,
                        b_up.reshape(1, UP_DIM))
    return _up_call(agg0, agg1, deg_dst, wf, bf)


# speedup vs baseline: 8.9434x; 1.0119x over previous
import functools

import jax
import jax.numpy as jnp
from jax import lax
from jax.experimental import pallas as pl
from jax.experimental.pallas import tpu as pltpu
from jax.experimental.pallas import tpu_sc as plsc

N = 50000
E = 800000
IN_DIM = 128
HID = 64
UP_DIM = 768

NPAD = 50176
STRIPE = NPAD // 16
ZCH = 784

EPB = 128
RPT = 396
EPAD = 16 * RPT * EPB
ROWS = EPAD // EPB
K = 3
NBLK = RPT // K

BLK = 400
GRID = N // BLK

_mesh = plsc.VectorSubcoreMesh(core_axis_name="c", subcore_axis_name="s")
_sc_params = pltpu.CompilerParams(use_tc_tiling_on_sc=False)


def _deg_body(src_rs, dst_rs, ones_hbm, zeros_hbm, out_src, out_dst,
              idxA, idxB, ones_v, semSA, semSB, hist):
    cid = lax.axis_index("c")
    tid = lax.axis_index("s")
    NP = NBLK // 2

    def run(eidx, out_ref):
        for j in range(STRIPE // ZCH):
            pltpu.sync_copy(zeros_hbm, hist.at[pl.ds(tid * STRIPE + j * ZCH, ZCH)])
        pltpu.sync_copy(ones_hbm, ones_v)
        plsc.subcore_barrier()
        base = tid * RPT

        def issue(b, idx, semS):
            pltpu.sync_copy(eidx.at[pl.ds(base + b * K, K)], idx)
            for k in range(K):
                pltpu.async_copy(ones_v, hist.at[idx.at[k]], semS, add=True)

        def wait_s(idx, semS):
            for k in range(K):
                pltpu.make_async_copy(ones_v, hist.at[idx.at[k]], semS).wait()

        issue(0, idxA, semSA)

        def body(sb, carry):
            b0 = 2 * sb

            @pl.when(sb > 0)
            def _():
                wait_s(idxB, semSB)

            issue(b0 + 1, idxB, semSB)

            @pl.when(sb + 1 < NP)
            def _():
                wait_s(idxA, semSA)
                issue(b0 + 2, idxA, semSA)

            return carry

        lax.fori_loop(0, NP, body, 0)
        wait_s(idxA, semSA)
        wait_s(idxB, semSB)
        plsc.subcore_barrier()
        sl = pl.ds(tid * STRIPE, STRIPE)
        pltpu.sync_copy(hist.at[sl], out_ref.at[sl])

    @pl.when(cid == 0)
    def _():
        run(src_rs, out_src)

    @pl.when(cid == 1)
    def _():
        run(dst_rs, out_dst)


_deg_call = functools.partial(
    pl.kernel,
    out_type=(
        jax.ShapeDtypeStruct((NPAD, 16), jnp.float32),
        jax.ShapeDtypeStruct((NPAD, 16), jnp.float32),
    ),
    mesh=_mesh,
    scratch_types=[
        pltpu.VMEM((K, EPB), jnp.int32),
        pltpu.VMEM((K, EPB), jnp.int32),
        pltpu.VMEM((EPB, 16), jnp.float32),
        pltpu.SemaphoreType.DMA,
        pltpu.SemaphoreType.DMA,
        pltpu.VMEM_SHARED((NPAD, 16), jnp.float32),
    ],
    compiler_params=_sc_params,
)(_deg_body)


def _agg_body(sd_rs, h0, h1, zeros_hbm, out0, out1,
              idxA, idxB, rowsA, rowsB, semA, semB, semSA, semSB, acc):
    cid = lax.axis_index("c")
    tid = lax.axis_index("s")
    NP = NBLK // 2

    def run(h_ref, out_ref):
        for j in range(STRIPE // ZCH):
            pltpu.sync_copy(zeros_hbm, acc.at[pl.ds(tid * STRIPE + j * ZCH, ZCH)])
        plsc.subcore_barrier()
        base = tid * RPT

        def issue(b, idx, rows, sem):
            pltpu.sync_copy(sd_rs.at[pl.ds(base + b * K, K)], idx)
            for k in range(K):
                pltpu.async_copy(h_ref.at[idx.at[k, 0]], rows.at[k], sem)

        def wait_g(idx, rows, sem):
            for k in range(K):
                pltpu.make_async_copy(h_ref.at[idx.at[k, 0]], rows.at[k], sem).wait()

        def scatter(idx, rows, semS):
            for k in range(K):
                pltpu.async_copy(rows.at[k], acc.at[idx.at[k, 1]], semS, add=True)

        def wait_s(idx, rows, semS):
            for k in range(K):
                pltpu.make_async_copy(rows.at[k], acc.at[idx.at[k, 1]], semS).wait()

        issue(0, idxA, rowsA, semA)

        def body(sb, carry):
            b0 = 2 * sb

            @pl.when(sb > 0)
            def _():
                wait_s(idxB, rowsB, semSB)

            issue(b0 + 1, idxB, rowsB, semB)
            wait_g(idxA, rowsA, semA)
            scatter(idxA, rowsA, semSA)

            @pl.when(sb + 1 < NP)
            def _():
                wait_s(idxA, rowsA, semSA)
                issue(b0 + 2, idxA, rowsA, semA)

            wait_g(idxB, rowsB, semB)
            scatter(idxB, rowsB, semSB)
            return carry

        lax.fori_loop(0, NP, body, 0)
        wait_s(idxA, rowsA, semSA)
        wait_s(idxB, rowsB, semSB)
        plsc.subcore_barrier()
        sl = pl.ds(tid * STRIPE, STRIPE)
        pltpu.sync_copy(acc.at[sl], out_ref.at[sl])

    @pl.when(cid == 0)
    def _():
        run(h0, out0)

    @pl.when(cid == 1)
    def _():
        run(h1, out1)


_agg_call = functools.partial(
    pl.kernel,
    out_type=(
        jax.ShapeDtypeStruct((NPAD, 32), jnp.float32),
        jax.ShapeDtypeStruct((NPAD, 32), jnp.float32),
    ),
    mesh=_mesh,
    scratch_types=[
        pltpu.VMEM((K, 2, EPB), jnp.int32),
        pltpu.VMEM((K, 2, EPB), jnp.int32),
        pltpu.VMEM((K, EPB, 32), jnp.float32),
        pltpu.VMEM((K, EPB, 32), jnp.float32),
        pltpu.SemaphoreType.DMA,
        pltpu.SemaphoreType.DMA,
        pltpu.SemaphoreType.DMA,
        pltpu.SemaphoreType.DMA,
        pltpu.VMEM_SHARED((NPAD, 32), jnp.float32),
    ],
    compiler_params=_sc_params,
)(_agg_body)


def _down_body(x_ref, w_ref, b_ref, d_ref, h0_ref, h1_ref):
    h = jnp.dot(x_ref[...], w_ref[...],
                preferred_element_type=jnp.float32,
                precision=lax.Precision.DEFAULT) + b_ref[...]
    norm = lax.rsqrt(jnp.maximum(d_ref[:, :1], 1.0))
    h = h * norm
    h0_ref[...] = h[:, :32]
    h1_ref[...] = h[:, 32:]


def _down_call(x, w, b, deg):
    return pl.pallas_call(
        _down_body,
        grid=(GRID,),
        in_specs=[
            pl.BlockSpec((BLK, IN_DIM), lambda i: (i, 0)),
            pl.BlockSpec((IN_DIM, HID), lambda i: (0, 0)),
            pl.BlockSpec((1, HID), lambda i: (0, 0)),
            pl.BlockSpec((BLK, 16), lambda i: (i, 0)),
        ],
        out_specs=(
            pl.BlockSpec((BLK, 32), lambda i: (i, 0)),
            pl.BlockSpec((BLK, 32), lambda i: (i, 0)),
        ),
        out_shape=(
            jax.ShapeDtypeStruct((N, 32), jnp.float32),
            jax.ShapeDtypeStruct((N, 32), jnp.float32),
        ),
    )(x, w, b, deg)


def _fuse_body(wg_ref, bg_ref, wu_ref, bu_ref, wf_ref, bf_ref):
    wf_ref[...] = jnp.dot(wg_ref[...], wu_ref[...],
                          preferred_element_type=jnp.float32,
                          precision=lax.Precision.HIGHEST)
    bf_ref[...] = jnp.dot(bg_ref[...], wu_ref[...],
                          preferred_element_type=jnp.float32,
                          precision=lax.Precision.HIGHEST) + bu_ref[...]


def _fuse_call(wg, bg, wu, bu):
    return pl.pallas_call(
        _fuse_body,
        out_shape=(
            jax.ShapeDtypeStruct((HID, UP_DIM), jnp.float32),
            jax.ShapeDtypeStruct((1, UP_DIM), jnp.float32),
        ),
    )(wg, bg, wu, bu)


def _up_body(a0_ref, a1_ref, d_ref, wf_ref, bf_ref, o_ref):
    a = jnp.concatenate([a0_ref[...], a1_ref[...]], axis=1)
    norm = lax.rsqrt(jnp.maximum(d_ref[:, :1], 1.0))
    a = a * norm
    o_ref[...] = jnp.dot(a, wf_ref[...],
                         preferred_element_type=jnp.float32,
                         precision=lax.Precision.DEFAULT) + bf_ref[...]


def _up_call(a0, a1, deg, wf, bf):
    return pl.pallas_call(
        _up_body,
        grid=(GRID,),
        in_specs=[
            pl.BlockSpec((BLK, 32), lambda i: (i, 0)),
            pl.BlockSpec((BLK, 32), lambda i: (i, 0)),
            pl.BlockSpec((BLK, 16), lambda i: (i, 0)),
            pl.BlockSpec((HID, UP_DIM), lambda i: (0, 0)),
            pl.BlockSpec((1, UP_DIM), lambda i: (0, 0)),
        ],
        out_specs=pl.BlockSpec((BLK, UP_DIM), lambda i: (i, 0)),
        out_shape=jax.ShapeDtypeStruct((N, UP_DIM), jnp.float32),
    )(a0, a1, deg, wf, bf)


def kernel(features, edge_index, W_down, b_down, W_gnn, b_gnn, W_up, b_up):
    src = edge_index[0]
    dst = edge_index[1]
    npad_extra = NPAD - N
    pad_n = EPAD - E
    pad_trash = N + (jnp.arange(pad_n, dtype=jnp.int32) % npad_extra)
    pad_low = jnp.arange(pad_n, dtype=jnp.int32) % EPB
    src_deg_rs = jnp.concatenate([src, pad_trash]).reshape(ROWS, EPB)
    src_gat_rs = jnp.concatenate([src, pad_low]).reshape(ROWS, EPB)
    dst_rs = jnp.concatenate([dst, pad_trash]).reshape(ROWS, EPB)
    sd_rs = jnp.stack([src_gat_rs, dst_rs], axis=1)

    ones16 = jnp.ones((EPB, 16), jnp.float32)
    zeros16 = jnp.zeros((ZCH, 16), jnp.float32)
    zeros32 = jnp.zeros((ZCH, 32), jnp.float32)

    deg_src, deg_dst = _deg_call(src_deg_rs, dst_rs, ones16, zeros16)

    h0, h1 = _down_call(features, W_down, b_down.reshape(1, HID), deg_src)

    agg0, agg1 = _agg_call(sd_rs, h0, h1, zeros32)

    wf, bf = _fuse_call(W_gnn, b_gnn.reshape(1, HID), W_up,
                        b_up.reshape(1, UP_DIM))
    return _up_call(agg0, agg1, deg_dst, wf, bf)


# speedup vs baseline: 10.7562x; 1.2027x over previous
import functools

import jax
import jax.numpy as jnp
from jax import lax
from jax.experimental import pallas as pl
from jax.experimental.pallas import tpu as pltpu
from jax.experimental.pallas import tpu_sc as plsc

N = 50000
E = 800000
IN_DIM = 128
HID = 64
UP_DIM = 768

NPAD = 50176
STRIPE = NPAD // 16
ZCH = 784

EPB = 128
RPT = 396
EPAD = 16 * RPT * EPB
ROWS = EPAD // EPB
K = 3
NBLK = RPT // K

BLK = 2000
GRID = N // BLK

_mesh = plsc.VectorSubcoreMesh(core_axis_name="c", subcore_axis_name="s")
_sc_params = pltpu.CompilerParams(use_tc_tiling_on_sc=False)


def _deg_body(src_rs, dst_rs, ones_hbm, zeros_hbm, out_src, out_dst,
              idxA, idxB, ones_v, semSA, semSB, hist):
    cid = lax.axis_index("c")
    tid = lax.axis_index("s")
    NP = NBLK // 2

    def run(eidx, out_ref):
        for j in range(STRIPE // ZCH):
            pltpu.sync_copy(zeros_hbm, hist.at[pl.ds(tid * STRIPE + j * ZCH, ZCH)])
        pltpu.sync_copy(ones_hbm, ones_v)
        plsc.subcore_barrier()
        base = tid * RPT

        def issue(b, idx, semS):
            pltpu.sync_copy(eidx.at[pl.ds(base + b * K, K)], idx)
            for k in range(K):
                pltpu.async_copy(ones_v, hist.at[idx.at[k]], semS, add=True)

        def wait_s(idx, semS):
            for k in range(K):
                pltpu.make_async_copy(ones_v, hist.at[idx.at[k]], semS).wait()

        issue(0, idxA, semSA)

        def body(sb, carry):
            b0 = 2 * sb

            @pl.when(sb > 0)
            def _():
                wait_s(idxB, semSB)

            issue(b0 + 1, idxB, semSB)

            @pl.when(sb + 1 < NP)
            def _():
                wait_s(idxA, semSA)
                issue(b0 + 2, idxA, semSA)

            return carry

        lax.fori_loop(0, NP, body, 0)
        wait_s(idxA, semSA)
        wait_s(idxB, semSB)
        plsc.subcore_barrier()
        sl = pl.ds(tid * STRIPE, STRIPE)
        pltpu.sync_copy(hist.at[sl], out_ref.at[sl])

    @pl.when(cid == 0)
    def _():
        run(src_rs, out_src)

    @pl.when(cid == 1)
    def _():
        run(dst_rs, out_dst)


_deg_call = functools.partial(
    pl.kernel,
    out_type=(
        jax.ShapeDtypeStruct((NPAD, 16), jnp.float32),
        jax.ShapeDtypeStruct((NPAD, 16), jnp.float32),
    ),
    mesh=_mesh,
    scratch_types=[
        pltpu.VMEM((K, EPB), jnp.int32),
        pltpu.VMEM((K, EPB), jnp.int32),
        pltpu.VMEM((EPB, 16), jnp.float32),
        pltpu.SemaphoreType.DMA,
        pltpu.SemaphoreType.DMA,
        pltpu.VMEM_SHARED((NPAD, 16), jnp.float32),
    ],
    compiler_params=_sc_params,
)(_deg_body)


def _agg_body(sd_rs, h0, h1, zeros_hbm, out0, out1,
              idxA, idxB, rowsA, rowsB, semA, semB, semSA, semSB, acc):
    cid = lax.axis_index("c")
    tid = lax.axis_index("s")
    NP = NBLK // 2

    def run(h_ref, out_ref):
        for j in range(STRIPE // ZCH):
            pltpu.sync_copy(zeros_hbm, acc.at[pl.ds(tid * STRIPE + j * ZCH, ZCH)])
        plsc.subcore_barrier()
        base = tid * RPT

        def issue(b, idx, rows, sem):
            pltpu.sync_copy(sd_rs.at[pl.ds(base + b * K, K)], idx)
            for k in range(K):
                pltpu.async_copy(h_ref.at[idx.at[k, 0]], rows.at[k], sem)

        def wait_g(idx, rows, sem):
            for k in range(K):
                pltpu.make_async_copy(h_ref.at[idx.at[k, 0]], rows.at[k], sem).wait()

        def scatter(idx, rows, semS):
            for k in range(K):
                pltpu.async_copy(rows.at[k], acc.at[idx.at[k, 1]], semS, add=True)

        def wait_s(idx, rows, semS):
            for k in range(K):
                pltpu.make_async_copy(rows.at[k], acc.at[idx.at[k, 1]], semS).wait()

        issue(0, idxA, rowsA, semA)

        def body(sb, carry):
            b0 = 2 * sb

            @pl.when(sb > 0)
            def _():
                wait_s(idxB, rowsB, semSB)

            issue(b0 + 1, idxB, rowsB, semB)
            wait_g(idxA, rowsA, semA)
            scatter(idxA, rowsA, semSA)

            @pl.when(sb + 1 < NP)
            def _():
                wait_s(idxA, rowsA, semSA)
                issue(b0 + 2, idxA, rowsA, semA)

            wait_g(idxB, rowsB, semB)
            scatter(idxB, rowsB, semSB)
            return carry

        lax.fori_loop(0, NP, body, 0)
        wait_s(idxA, rowsA, semSA)
        wait_s(idxB, rowsB, semSB)
        plsc.subcore_barrier()
        sl = pl.ds(tid * STRIPE, STRIPE)
        pltpu.sync_copy(acc.at[sl], out_ref.at[sl])

    @pl.when(cid == 0)
    def _():
        run(h0, out0)

    @pl.when(cid == 1)
    def _():
        run(h1, out1)


_agg_call = functools.partial(
    pl.kernel,
    out_type=(
        jax.ShapeDtypeStruct((NPAD, 32), jnp.float32),
        jax.ShapeDtypeStruct((NPAD, 32), jnp.float32),
    ),
    mesh=_mesh,
    scratch_types=[
        pltpu.VMEM((K, 2, EPB), jnp.int32),
        pltpu.VMEM((K, 2, EPB), jnp.int32),
        pltpu.VMEM((K, EPB, 32), jnp.float32),
        pltpu.VMEM((K, EPB, 32), jnp.float32),
        pltpu.SemaphoreType.DMA,
        pltpu.SemaphoreType.DMA,
        pltpu.SemaphoreType.DMA,
        pltpu.SemaphoreType.DMA,
        pltpu.VMEM_SHARED((NPAD, 32), jnp.float32),
    ],
    compiler_params=_sc_params,
)(_agg_body)


def _down_body(x_ref, w_ref, b_ref, d_ref, h0_ref, h1_ref):
    h = jnp.dot(x_ref[...], w_ref[...],
                preferred_element_type=jnp.float32,
                precision=lax.Precision.DEFAULT) + b_ref[...]
    norm = lax.rsqrt(jnp.maximum(d_ref[:, :1], 1.0))
    h = h * norm
    h0_ref[...] = h[:, :32]
    h1_ref[...] = h[:, 32:]


def _down_call(x, w, b, deg):
    return pl.pallas_call(
        _down_body,
        grid=(GRID,),
        in_specs=[
            pl.BlockSpec((BLK, IN_DIM), lambda i: (i, 0)),
            pl.BlockSpec((IN_DIM, HID), lambda i: (0, 0)),
            pl.BlockSpec((1, HID), lambda i: (0, 0)),
            pl.BlockSpec((BLK, 16), lambda i: (i, 0)),
        ],
        out_specs=(
            pl.BlockSpec((BLK, 32), lambda i: (i, 0)),
            pl.BlockSpec((BLK, 32), lambda i: (i, 0)),
        ),
        out_shape=(
            jax.ShapeDtypeStruct((N, 32), jnp.float32),
            jax.ShapeDtypeStruct((N, 32), jnp.float32),
        ),
    )(x, w, b, deg)


def _fuse_body(wg_ref, bg_ref, wu_ref, bu_ref, wf_ref, bf_ref):
    wf_ref[...] = jnp.dot(wg_ref[...], wu_ref[...],
                          preferred_element_type=jnp.float32,
                          precision=lax.Precision.HIGHEST)
    bf_ref[...] = jnp.dot(bg_ref[...], wu_ref[...],
                          preferred_element_type=jnp.float32,
                          precision=lax.Precision.HIGHEST) + bu_ref[...]


def _fuse_call(wg, bg, wu, bu):
    return pl.pallas_call(
        _fuse_body,
        out_shape=(
            jax.ShapeDtypeStruct((HID, UP_DIM), jnp.float32),
            jax.ShapeDtypeStruct((1, UP_DIM), jnp.float32),
        ),
    )(wg, bg, wu, bu)


def _up_body(a0_ref, a1_ref, d_ref, wf_ref, bf_ref, o_ref):
    a = jnp.concatenate([a0_ref[...], a1_ref[...]], axis=1)
    norm = lax.rsqrt(jnp.maximum(d_ref[:, :1], 1.0))
    a = a * norm
    o_ref[...] = jnp.dot(a, wf_ref[...],
                         preferred_element_type=jnp.float32,
                         precision=lax.Precision.DEFAULT) + bf_ref[...]


def _up_call(a0, a1, deg, wf, bf):
    return pl.pallas_call(
        _up_body,
        grid=(GRID,),
        in_specs=[
            pl.BlockSpec((BLK, 32), lambda i: (i, 0)),
            pl.BlockSpec((BLK, 32), lambda i: (i, 0)),
            pl.BlockSpec((BLK, 16), lambda i: (i, 0)),
            pl.BlockSpec((HID, UP_DIM), lambda i: (0, 0)),
            pl.BlockSpec((1, UP_DIM), lambda i: (0, 0)),
        ],
        out_specs=pl.BlockSpec((BLK, UP_DIM), lambda i: (i, 0)),
        out_shape=jax.ShapeDtypeStruct((N, UP_DIM), jnp.float32),
    )(a0, a1, deg, wf, bf)


def kernel(features, edge_index, W_down, b_down, W_gnn, b_gnn, W_up, b_up):
    src = edge_index[0]
    dst = edge_index[1]
    npad_extra = NPAD - N
    pad_n = EPAD - E
    pad_trash = N + (jnp.arange(pad_n, dtype=jnp.int32) % npad_extra)
    pad_low = jnp.arange(pad_n, dtype=jnp.int32) % EPB
    src_deg_rs = jnp.concatenate([src, pad_trash]).reshape(ROWS, EPB)
    src_gat_rs = jnp.concatenate([src, pad_low]).reshape(ROWS, EPB)
    dst_rs = jnp.concatenate([dst, pad_trash]).reshape(ROWS, EPB)
    sd_rs = jnp.stack([src_gat_rs, dst_rs], axis=1)

    ones16 = jnp.ones((EPB, 16), jnp.float32)
    zeros16 = jnp.zeros((ZCH, 16), jnp.float32)
    zeros32 = jnp.zeros((ZCH, 32), jnp.float32)

    deg_src, deg_dst = _deg_call(src_deg_rs, dst_rs, ones16, zeros16)

    h0, h1 = _down_call(features, W_down, b_down.reshape(1, HID), deg_src)

    agg0, agg1 = _agg_call(sd_rs, h0, h1, zeros32)

    wf, bf = _fuse_call(W_gnn, b_gnn.reshape(1, HID), W_up,
                        b_up.reshape(1, UP_DIM))
    return _up_call(agg0, agg1, deg_dst, wf, bf)


# speedup vs baseline: 11.0019x; 1.0228x over previous
import functools

import jax
import jax.numpy as jnp
from jax import lax
from jax.experimental import pallas as pl
from jax.experimental.pallas import tpu as pltpu
from jax.experimental.pallas import tpu_sc as plsc

N = 50000
E = 800000
IN_DIM = 128
HID = 64
UP_DIM = 768

NPAD = 50176
STRIPE = NPAD // 16
ZCH = 784

EPB = 128
RPT = 396
EPAD = 16 * RPT * EPB
ROWS = EPAD // EPB
K = 3
NBLK = RPT // K

BLK = 5000
GRID = N // BLK

_mesh = plsc.VectorSubcoreMesh(core_axis_name="c", subcore_axis_name="s")
_sc_params = pltpu.CompilerParams(use_tc_tiling_on_sc=False)


def _deg_body(src_rs, dst_rs, ones_hbm, zeros_hbm, out_src, out_dst,
              idxA, idxB, ones_v, semSA, semSB, hist):
    cid = lax.axis_index("c")
    tid = lax.axis_index("s")
    NP = NBLK // 2

    def run(eidx, out_ref):
        for j in range(STRIPE // ZCH):
            pltpu.sync_copy(zeros_hbm, hist.at[pl.ds(tid * STRIPE + j * ZCH, ZCH)])
        pltpu.sync_copy(ones_hbm, ones_v)
        plsc.subcore_barrier()
        base = tid * RPT

        def issue(b, idx, semS):
            pltpu.sync_copy(eidx.at[pl.ds(base + b * K, K)], idx)
            for k in range(K):
                pltpu.async_copy(ones_v, hist.at[idx.at[k]], semS, add=True)

        def wait_s(idx, semS):
            for k in range(K):
                pltpu.make_async_copy(ones_v, hist.at[idx.at[k]], semS).wait()

        issue(0, idxA, semSA)

        def body(sb, carry):
            b0 = 2 * sb

            @pl.when(sb > 0)
            def _():
                wait_s(idxB, semSB)

            issue(b0 + 1, idxB, semSB)

            @pl.when(sb + 1 < NP)
            def _():
                wait_s(idxA, semSA)
                issue(b0 + 2, idxA, semSA)

            return carry

        lax.fori_loop(0, NP, body, 0)
        wait_s(idxA, semSA)
        wait_s(idxB, semSB)
        plsc.subcore_barrier()
        sl = pl.ds(tid * STRIPE, STRIPE)
        pltpu.sync_copy(hist.at[sl], out_ref.at[sl])

    @pl.when(cid == 0)
    def _():
        run(src_rs, out_src)

    @pl.when(cid == 1)
    def _():
        run(dst_rs, out_dst)


_deg_call = functools.partial(
    pl.kernel,
    out_type=(
        jax.ShapeDtypeStruct((NPAD, 16), jnp.float32),
        jax.ShapeDtypeStruct((NPAD, 16), jnp.float32),
    ),
    mesh=_mesh,
    scratch_types=[
        pltpu.VMEM((K, EPB), jnp.int32),
        pltpu.VMEM((K, EPB), jnp.int32),
        pltpu.VMEM((EPB, 16), jnp.float32),
        pltpu.SemaphoreType.DMA,
        pltpu.SemaphoreType.DMA,
        pltpu.VMEM_SHARED((NPAD, 16), jnp.float32),
    ],
    compiler_params=_sc_params,
)(_deg_body)


def _agg_body(sd_rs, h0, h1, zeros_hbm, out0, out1,
              idxA, idxB, rowsA, rowsB, semA, semB, semSA, semSB, acc):
    cid = lax.axis_index("c")
    tid = lax.axis_index("s")
    NP = NBLK // 2

    def run(h_ref, out_ref):
        for j in range(STRIPE // ZCH):
            pltpu.sync_copy(zeros_hbm, acc.at[pl.ds(tid * STRIPE + j * ZCH, ZCH)])
        plsc.subcore_barrier()
        base = tid * RPT

        def issue(b, idx, rows, sem):
            pltpu.sync_copy(sd_rs.at[pl.ds(base + b * K, K)], idx)
            for k in range(K):
                pltpu.async_copy(h_ref.at[idx.at[k, 0]], rows.at[k], sem)

        def wait_g(idx, rows, sem):
            for k in range(K):
                pltpu.make_async_copy(h_ref.at[idx.at[k, 0]], rows.at[k], sem).wait()

        def scatter(idx, rows, semS):
            for k in range(K):
                pltpu.async_copy(rows.at[k], acc.at[idx.at[k, 1]], semS, add=True)

        def wait_s(idx, rows, semS):
            for k in range(K):
                pltpu.make_async_copy(rows.at[k], acc.at[idx.at[k, 1]], semS).wait()

        issue(0, idxA, rowsA, semA)

        def body(sb, carry):
            b0 = 2 * sb

            @pl.when(sb > 0)
            def _():
                wait_s(idxB, rowsB, semSB)

            issue(b0 + 1, idxB, rowsB, semB)
            wait_g(idxA, rowsA, semA)
            scatter(idxA, rowsA, semSA)

            @pl.when(sb + 1 < NP)
            def _():
                wait_s(idxA, rowsA, semSA)
                issue(b0 + 2, idxA, rowsA, semA)

            wait_g(idxB, rowsB, semB)
            scatter(idxB, rowsB, semSB)
            return carry

        lax.fori_loop(0, NP, body, 0)
        wait_s(idxA, rowsA, semSA)
        wait_s(idxB, rowsB, semSB)
        plsc.subcore_barrier()
        sl = pl.ds(tid * STRIPE, STRIPE)
        pltpu.sync_copy(acc.at[sl], out_ref.at[sl])

    @pl.when(cid == 0)
    def _():
        run(h0, out0)

    @pl.when(cid == 1)
    def _():
        run(h1, out1)


_agg_call = functools.partial(
    pl.kernel,
    out_type=(
        jax.ShapeDtypeStruct((NPAD, 32), jnp.float32),
        jax.ShapeDtypeStruct((NPAD, 32), jnp.float32),
    ),
    mesh=_mesh,
    scratch_types=[
        pltpu.VMEM((K, 2, EPB), jnp.int32),
        pltpu.VMEM((K, 2, EPB), jnp.int32),
        pltpu.VMEM((K, EPB, 32), jnp.float32),
        pltpu.VMEM((K, EPB, 32), jnp.float32),
        pltpu.SemaphoreType.DMA,
        pltpu.SemaphoreType.DMA,
        pltpu.SemaphoreType.DMA,
        pltpu.SemaphoreType.DMA,
        pltpu.VMEM_SHARED((NPAD, 32), jnp.float32),
    ],
    compiler_params=_sc_params,
)(_agg_body)


def _down_body(x_ref, w_ref, b_ref, d_ref, h0_ref, h1_ref):
    h = jnp.dot(x_ref[...], w_ref[...],
                preferred_element_type=jnp.float32,
                precision=lax.Precision.DEFAULT) + b_ref[...]
    norm = lax.rsqrt(jnp.maximum(d_ref[:, :1], 1.0))
    h = h * norm
    h0_ref[...] = h[:, :32]
    h1_ref[...] = h[:, 32:]


def _down_call(x, w, b, deg):
    return pl.pallas_call(
        _down_body,
        grid=(GRID,),
        in_specs=[
            pl.BlockSpec((BLK, IN_DIM), lambda i: (i, 0)),
            pl.BlockSpec((IN_DIM, HID), lambda i: (0, 0)),
            pl.BlockSpec((1, HID), lambda i: (0, 0)),
            pl.BlockSpec((BLK, 16), lambda i: (i, 0)),
        ],
        out_specs=(
            pl.BlockSpec((BLK, 32), lambda i: (i, 0)),
            pl.BlockSpec((BLK, 32), lambda i: (i, 0)),
        ),
        out_shape=(
            jax.ShapeDtypeStruct((N, 32), jnp.float32),
            jax.ShapeDtypeStruct((N, 32), jnp.float32),
        ),
    )(x, w, b, deg)


def _fuse_body(wg_ref, bg_ref, wu_ref, bu_ref, wf_ref, bf_ref):
    wf_ref[...] = jnp.dot(wg_ref[...], wu_ref[...],
                          preferred_element_type=jnp.float32,
                          precision=lax.Precision.HIGHEST)
    bf_ref[...] = jnp.dot(bg_ref[...], wu_ref[...],
                          preferred_element_type=jnp.float32,
                          precision=lax.Precision.HIGHEST) + bu_ref[...]


def _fuse_call(wg, bg, wu, bu):
    return pl.pallas_call(
        _fuse_body,
        out_shape=(
            jax.ShapeDtypeStruct((HID, UP_DIM), jnp.float32),
            jax.ShapeDtypeStruct((1, UP_DIM), jnp.float32),
        ),
    )(wg, bg, wu, bu)


def _up_body(a0_ref, a1_ref, d_ref, wf_ref, bf_ref, o_ref):
    a = jnp.concatenate([a0_ref[...], a1_ref[...]], axis=1)
    norm = lax.rsqrt(jnp.maximum(d_ref[:, :1], 1.0))
    a = a * norm
    o_ref[...] = jnp.dot(a, wf_ref[...],
                         preferred_element_type=jnp.float32,
                         precision=lax.Precision.DEFAULT) + bf_ref[...]


def _up_call(a0, a1, deg, wf, bf):
    return pl.pallas_call(
        _up_body,
        grid=(GRID,),
        in_specs=[
            pl.BlockSpec((BLK, 32), lambda i: (i, 0)),
            pl.BlockSpec((BLK, 32), lambda i: (i, 0)),
            pl.BlockSpec((BLK, 16), lambda i: (i, 0)),
            pl.BlockSpec((HID, UP_DIM), lambda i: (0, 0)),
            pl.BlockSpec((1, UP_DIM), lambda i: (0, 0)),
        ],
        out_specs=pl.BlockSpec((BLK, UP_DIM), lambda i: (i, 0)),
        out_shape=jax.ShapeDtypeStruct((N, UP_DIM), jnp.float32),
    )(a0, a1, deg, wf, bf)


def kernel(features, edge_index, W_down, b_down, W_gnn, b_gnn, W_up, b_up):
    src = edge_index[0]
    dst = edge_index[1]
    npad_extra = NPAD - N
    pad_n = EPAD - E
    pad_trash = N + (jnp.arange(pad_n, dtype=jnp.int32) % npad_extra)
    pad_low = jnp.arange(pad_n, dtype=jnp.int32) % EPB
    src_deg_rs = jnp.concatenate([src, pad_trash]).reshape(ROWS, EPB)
    src_gat_rs = jnp.concatenate([src, pad_low]).reshape(ROWS, EPB)
    dst_rs = jnp.concatenate([dst, pad_trash]).reshape(ROWS, EPB)
    sd_rs = jnp.stack([src_gat_rs, dst_rs], axis=1)

    ones16 = jnp.ones((EPB, 16), jnp.float32)
    zeros16 = jnp.zeros((ZCH, 16), jnp.float32)
    zeros32 = jnp.zeros((ZCH, 32), jnp.float32)

    deg_src, deg_dst = _deg_call(src_deg_rs, dst_rs, ones16, zeros16)

    h0, h1 = _down_call(features, W_down, b_down.reshape(1, HID), deg_src)

    agg0, agg1 = _agg_call(sd_rs, h0, h1, zeros32)

    wf, bf = _fuse_call(W_gnn, b_gnn.reshape(1, HID), W_up,
                        b_up.reshape(1, UP_DIM))
    return _up_call(agg0, agg1, deg_dst, wf, bf)
